# baseline (speedup 1.0000x reference)
import functools

import jax
import jax.numpy as jnp
from jax import lax
from jax.experimental import pallas as pl
from jax.experimental.pallas import tpu as pltpu
from jax.experimental.pallas import tpu_sc as plsc

_KNN = 20
_H = 16
_N = 256
_L = _N * _KNN
_NC = 2
_NS = 16
_NW = _NC * _NS
_ROWS_PER_W = _N // _NW


def _sc_gather_body(table_hbm, dv_hbm, g_hbm, idx_v, rows_v, sem):
    wid = lax.axis_index("s") * _NC + lax.axis_index("c")
    base = wid * _ROWS_PER_W
    pltpu.sync_copy(dv_hbm.at[pl.ds(base, _ROWS_PER_W)], idx_v)
    pltpu.async_copy(table_hbm.at[idx_v], rows_v, sem).wait()
    pltpu.sync_copy(rows_v, g_hbm.at[pl.ds(base, _ROWS_PER_W)])


@functools.cache
def _sc_gather():
    return pl.kernel(
        _sc_gather_body,
        compiler_params=pltpu.CompilerParams(
            use_tc_tiling_on_sc=False, needs_layout_passes=False),
        mesh=plsc.VectorSubcoreMesh(
            core_axis_name="c", subcore_axis_name="s",
            num_cores=_NC, num_subcores=_NS),
        out_type=jax.ShapeDtypeStruct((_N, _H), jnp.float32),
        scratch_types=[
            pltpu.VMEM((_ROWS_PER_W,), jnp.int32),
            pltpu.VMEM((_ROWS_PER_W, _H), jnp.float32),
            pltpu.SemaphoreType.DMA,
        ],
    )


_BLK_A = 8


def _tc_mask_body(cm_ref, b_ref):
    jj = lax.broadcasted_iota(jnp.int32, (_N, _L), 1) // _KNN
    aa = lax.broadcasted_iota(jnp.int32, (_N, _L), 0)
    ef = (jj == aa).astype(jnp.float32)
    m = lax.dot_general(cm_ref[...], ef, (((1,), (0,)), ((), ())),
                        preferred_element_type=jnp.float32)
    b_ref[...] = jnp.concatenate([1.0 - m, m], axis=0)


def _tc_mask(cm2, interpret=False):
    return pl.pallas_call(
        _tc_mask_body,
        out_shape=jax.ShapeDtypeStruct((2, _L), jnp.float32),
        interpret=interpret,
    )(cm2)


def _tc_expand_body(g_ref, b_ref, mt_ref, out_ref):
    g = g_ref[...]
    mt = mt_ref[...]
    q = lax.dot_general(g, mt, (((1,), (1,)), ((), ())),
                        preferred_element_type=jnp.float32)
    rr = lax.broadcasted_iota(jnp.int32, (_BLK_A * _KNN, _BLK_A), 0) // _KNN
    pp = lax.broadcasted_iota(jnp.int32, (_BLK_A * _KNN, _BLK_A), 1)
    e = (rr == pp).astype(jnp.float32)
    qe = lax.dot_general(e, q, (((1,), (0,)), ((), ())),
                         preferred_element_type=jnp.float32)
    vals = lax.dot_general(qe, b_ref[...], (((1,), (0,)), ((), ())),
                           preferred_element_type=jnp.float32)
    out_ref[...] = vals.reshape(1, 1, _BLK_A * _KNN, _L)


def _tc_expand(g, m, mask_table, interpret=False):
    return pl.pallas_call(
        _tc_expand_body,
        grid=(_N // _BLK_A,),
        in_specs=[
            pl.BlockSpec((_BLK_A, _H), lambda t: (t, 0)),
            pl.BlockSpec((2, _L), lambda t: (0, 0)),
            pl.BlockSpec((2, _H), lambda t: (0, 0)),
        ],
        out_specs=pl.BlockSpec((1, 1, _BLK_A * _KNN, _L), lambda t: (0, 0, t, 0)),
        out_shape=jax.ShapeDtypeStruct((1, 1, _L, _L), jnp.float32),
        interpret=interpret,
    )(g, m, mask_table)


@jax.jit
def kernel(c_mask, degrees_vec, edge_table, mask_table):
    g = _sc_gather()(edge_table, degrees_vec)
    cm2 = c_mask.astype(jnp.float32).reshape(1, _N)
    b = _tc_mask(cm2)
    return _tc_expand(g, b, mask_table)


# speedup vs baseline: 1.1433x; 1.1433x over previous
import functools

import jax
import jax.numpy as jnp
from jax import lax
from jax.experimental import pallas as pl
from jax.experimental.pallas import tpu as pltpu
from jax.experimental.pallas import tpu_sc as plsc

_KNN = 20
_H = 16
_N = 256
_L = _N * _KNN
_NC = 2
_NS = 16
_NW = _NC * _NS
_ROWS_PER_W = _N // _NW


def _sc_gather_body(table_hbm, dv_hbm, g_hbm, idx_v, rows_v, sem):
    wid = lax.axis_index("s") * _NC + lax.axis_index("c")
    base = wid * _ROWS_PER_W
    pltpu.sync_copy(dv_hbm.at[pl.ds(base, _ROWS_PER_W)], idx_v)
    pltpu.async_copy(table_hbm.at[idx_v], rows_v, sem).wait()
    pltpu.sync_copy(rows_v, g_hbm.at[pl.ds(base, _ROWS_PER_W)])


@functools.cache
def _sc_gather():
    return pl.kernel(
        _sc_gather_body,
        compiler_params=pltpu.CompilerParams(
            use_tc_tiling_on_sc=False, needs_layout_passes=False),
        mesh=plsc.VectorSubcoreMesh(
            core_axis_name="c", subcore_axis_name="s",
            num_cores=_NC, num_subcores=_NS),
        out_type=jax.ShapeDtypeStruct((_N, _H), jnp.float32),
        scratch_types=[
            pltpu.VMEM((_ROWS_PER_W,), jnp.int32),
            pltpu.VMEM((_ROWS_PER_W, _H), jnp.float32),
            pltpu.SemaphoreType.DMA,
        ],
    )


_BLK_A = 16


def _tc_mask_body(cm_ref, b_ref):
    jj = lax.broadcasted_iota(jnp.int32, (_N, _L), 1) // _KNN
    aa = lax.broadcasted_iota(jnp.int32, (_N, _L), 0)
    ef = (jj == aa).astype(jnp.float32)
    m = lax.dot_general(cm_ref[...], ef, (((1,), (0,)), ((), ())),
                        preferred_element_type=jnp.float32)
    b_ref[...] = jnp.concatenate([1.0 - m, m], axis=0)


def _tc_mask(cm2, interpret=False):
    return pl.pallas_call(
        _tc_mask_body,
        out_shape=jax.ShapeDtypeStruct((2, _L), jnp.float32),
        interpret=interpret,
    )(cm2)


def _tc_expand_body(g_ref, b_ref, mt_ref, out_ref):
    g = g_ref[...]
    mt = mt_ref[...]
    q = lax.dot_general(g, mt, (((1,), (1,)), ((), ())),
                        preferred_element_type=jnp.float32)
    rr = lax.broadcasted_iota(jnp.int32, (_BLK_A * _KNN, _BLK_A), 0) // _KNN
    pp = lax.broadcasted_iota(jnp.int32, (_BLK_A * _KNN, _BLK_A), 1)
    e = (rr == pp).astype(jnp.float32)
    qe = lax.dot_general(e, q, (((1,), (0,)), ((), ())),
                         preferred_element_type=jnp.float32)
    vals = lax.dot_general(qe, b_ref[...], (((1,), (0,)), ((), ())),
                           preferred_element_type=jnp.float32)
    out_ref[...] = vals.reshape(1, 1, _BLK_A * _KNN, _L)


def _tc_expand(g, m, mask_table, interpret=False):
    return pl.pallas_call(
        _tc_expand_body,
        grid=(_N // _BLK_A,),
        in_specs=[
            pl.BlockSpec((_BLK_A, _H), lambda t: (t, 0)),
            pl.BlockSpec((2, _L), lambda t: (0, 0)),
            pl.BlockSpec((2, _H), lambda t: (0, 0)),
        ],
        out_specs=pl.BlockSpec((1, 1, _BLK_A * _KNN, _L), lambda t: (0, 0, t, 0)),
        out_shape=jax.ShapeDtypeStruct((1, 1, _L, _L), jnp.float32),
        interpret=interpret,
    )(g, m, mask_table)


@jax.jit
def kernel(c_mask, degrees_vec, edge_table, mask_table):
    g = _sc_gather()(edge_table, degrees_vec)
    cm2 = c_mask.astype(jnp.float32).reshape(1, _N)
    b = _tc_mask(cm2)
    return _tc_expand(g, b, mask_table)
